# baseline (speedup 1.0000x reference)
import jax
import jax.numpy as jnp
from jax.experimental import pallas as pl
from jax.experimental.pallas import tpu as pltpu

_LANES = 128
_CHUNKS = 8


def _router_kernel(src, dst, tag_ref, load_ref, sem):
    n = src.shape[0]
    chunk = n // _CHUNKS
    for i in range(_CHUNKS):
        pltpu.make_async_copy(
            src.at[pl.ds(i * chunk, chunk), :],
            dst.at[pl.ds(i * chunk, chunk), :],
            sem.at[i],
        ).start()
    rows = tag_ref.shape[0]
    row = jax.lax.broadcasted_iota(jnp.int32, (rows, _LANES), 0)
    col = jax.lax.broadcasted_iota(jnp.int32, (rows, _LANES), 1)
    tag_ref[...] = row * _LANES + col
    load_ref[...] = jnp.full((1, 1), n, jnp.int32)
    for i in range(_CHUNKS):
        pltpu.make_async_copy(
            src.at[pl.ds(i * chunk, chunk), :],
            dst.at[pl.ds(i * chunk, chunk), :],
            sem.at[i],
        ).wait()


def kernel(in_flow):
    n, d = in_flow.shape
    rows = n // _LANES
    out, tag2d, load = pl.pallas_call(
        _router_kernel,
        in_specs=[pl.BlockSpec(memory_space=pl.ANY)],
        out_specs=(
            pl.BlockSpec(memory_space=pl.ANY),
            pl.BlockSpec(memory_space=pltpu.VMEM),
            pl.BlockSpec(memory_space=pltpu.VMEM),
        ),
        out_shape=(
            jax.ShapeDtypeStruct((n, d), in_flow.dtype),
            jax.ShapeDtypeStruct((rows, _LANES), jnp.int32),
            jax.ShapeDtypeStruct((1, 1), jnp.int32),
        ),
        scratch_shapes=[pltpu.SemaphoreType.DMA((_CHUNKS,))],
    )(in_flow)
    return (out, tag2d.reshape(n, 1), load.reshape(()))
